# baseline (speedup 1.0000x reference)
import jax
import jax.numpy as jnp
from jax.experimental import pallas as pl
from jax.experimental.pallas import tpu as pltpu

VOCAB_SIZE = 25107
EMB_DIM = 100
CTX_LEN = 5
BATCH = 16
H1 = 300
H2 = 300

VOCAB_TILE = 2048
NUM_FULL_TILES = VOCAB_SIZE // VOCAB_TILE
TAIL = VOCAB_SIZE - NUM_FULL_TILES * VOCAB_TILE
NBUF = 6
LANE = 128


def _mlp_kernel(xt_smem, xt_vmem, embt_hbm, w1t_ref, b1_ref, w2_ref, b2_ref,
                w3_hbm, b3_ref, out_hbm, wbuf, obuf, otail, bufs, tail_buf,
                gsem, osem, sems, tail_sem):
    def start_fetch(i):
        pltpu.make_async_copy(
            w3_hbm.at[:, pl.ds(i * VOCAB_TILE, VOCAB_TILE)],
            bufs.at[i % NBUF],
            sems.at[i % NBUF],
        ).start()

    tail_copy = pltpu.make_async_copy(
        w3_hbm.at[:, pl.ds(NUM_FULL_TILES * VOCAB_TILE, TAIL)],
        tail_buf,
        tail_sem,
    )
    tail_copy.start()
    for i in range(NBUF):
        start_fetch(i)
    out_copies = []

    gathers = []
    for b in range(BATCH):
        for c in range(CTX_LEN):
            win = (xt_smem[c, b] // LANE) * LANE
            g = pltpu.make_async_copy(
                embt_hbm.at[:, pl.ds(win, LANE)],
                wbuf.at[b, c], gsem)
            g.start()
            gathers.append(g)
    for g in gathers:
        g.wait()

    offs = jnp.transpose(xt_vmem[...], (1, 0)) % LANE
    lane_iota = jax.lax.broadcasted_iota(jnp.int32, (BATCH, 1, LANE), 2)

    w1t = w1t_ref[...]
    h1 = b1_ref[...][None, :]
    for c in range(CTX_LEN):
        sel = (lane_iota == offs[:, c][:, None, None]).astype(jnp.float32)
        e_c = jnp.sum(wbuf[:, c] * sel, axis=2)
        h1 = h1 + jax.lax.dot_general(
            e_c, w1t[:, c * EMB_DIM:(c + 1) * EMB_DIM],
            (((1,), (1,)), ((), ())),
            preferred_element_type=jnp.float32)
    h1 = jnp.maximum(h1, 0.0)
    h2 = jnp.maximum(
        jnp.dot(h1, w2_ref[...],
                preferred_element_type=jnp.float32) + b2_ref[...][None, :],
        0.0)

    for i in range(NUM_FULL_TILES):
        pltpu.make_async_copy(
            w3_hbm.at[:, pl.ds(i * VOCAB_TILE, VOCAB_TILE)],
            bufs.at[i % NBUF],
            sems.at[i % NBUF],
        ).wait()
        tile = jnp.dot(h2, bufs[i % NBUF],
                       preferred_element_type=jnp.float32)
        if i + NBUF < NUM_FULL_TILES:
            start_fetch(i + NBUF)
        if i >= 2:
            out_copies[i - 2].wait()
        obuf[i % 2] = tile + b3_ref[pl.ds(i * VOCAB_TILE, VOCAB_TILE)][None, :]
        oc = pltpu.make_async_copy(
            obuf.at[i % 2], out_hbm.at[:, pl.ds(i * VOCAB_TILE, VOCAB_TILE)],
            osem)
        oc.start()
        out_copies.append(oc)

    tail_copy.wait()
    base = NUM_FULL_TILES * VOCAB_TILE
    tail = jnp.dot(h2, tail_buf[...], preferred_element_type=jnp.float32)
    otail[...] = tail + b3_ref[pl.ds(base, TAIL)][None, :]
    tc2 = pltpu.make_async_copy(otail, out_hbm.at[:, pl.ds(base, TAIL)], osem)
    tc2.start()
    out_copies[NUM_FULL_TILES - 2].wait()
    out_copies[NUM_FULL_TILES - 1].wait()
    tc2.wait()


def kernel(x, emb, W1, b1, W2, b2, W3, b3):
    return pl.pallas_call(
        _mlp_kernel,
        in_specs=[
            pl.BlockSpec(memory_space=pltpu.SMEM),
            pl.BlockSpec(memory_space=pltpu.VMEM),
            pl.BlockSpec(memory_space=pl.ANY),
            pl.BlockSpec(memory_space=pltpu.VMEM),
            pl.BlockSpec(memory_space=pltpu.VMEM),
            pl.BlockSpec(memory_space=pltpu.VMEM),
            pl.BlockSpec(memory_space=pltpu.VMEM),
            pl.BlockSpec(memory_space=pl.ANY),
            pl.BlockSpec(memory_space=pltpu.VMEM),
        ],
        out_specs=pl.BlockSpec(memory_space=pl.ANY),
        out_shape=jax.ShapeDtypeStruct((BATCH, VOCAB_SIZE), jnp.float32),
        scratch_shapes=[
            pltpu.VMEM((BATCH, CTX_LEN, EMB_DIM, LANE), jnp.float32),
            pltpu.VMEM((2, BATCH, VOCAB_TILE), jnp.float32),
            pltpu.VMEM((BATCH, TAIL), jnp.float32),
            pltpu.VMEM((NBUF, H2, VOCAB_TILE), jnp.float32),
            pltpu.VMEM((H2, TAIL), jnp.float32),
            pltpu.SemaphoreType.DMA,
            pltpu.SemaphoreType.DMA,
            pltpu.SemaphoreType.DMA((NBUF,)),
            pltpu.SemaphoreType.DMA,
        ],
    )(x.T, x.T, emb.T, W1.T, b1, W2, b2, W3, b3)
